# baseline (speedup 1.0000x reference)
import functools
import jax
import jax.numpy as jnp
from jax import lax
from jax.experimental import pallas as pl
from jax.experimental.pallas import tpu as pltpu
from jax.experimental.pallas import tpu_sc as plsc

_HIDDEN = 2048
_K = 16


def _make_sc_kernel(n_total, nc, ns):
    nw = nc * ns
    b_per_w = n_total // nw
    ch = b_per_w // _K

    mesh = plsc.VectorSubcoreMesh(
        core_axis_name="c", subcore_axis_name="s", num_cores=nc, num_subcores=ns
    )

    @functools.partial(
        pl.kernel,
        out_type=jax.ShapeDtypeStruct((n_total, _HIDDEN), jnp.float32),
        mesh=mesh,
        scratch_types=[
            pltpu.VMEM((18, _HIDDEN), jnp.float32),
            pltpu.VMEM((ch, _K), jnp.int32),
            pltpu.SemaphoreType.DMA,
        ],
    )
    def k(table_hbm, idx_hbm, out_hbm, pairs_v, idx_v, sem):
        wid = lax.axis_index("s") * nc + lax.axis_index("c")
        base = wid * b_per_w
        pltpu.sync_copy(idx_hbm.at[wid], idx_v)
        for a in range(3):
            for b in range(3):
                p = 3 * a + b
                pltpu.async_copy(
                    table_hbm.at[pl.ds(a, 1)], pairs_v.at[pl.ds(2 * p, 1)], sem
                )
                pltpu.async_copy(
                    table_hbm.at[pl.ds(b, 1)], pairs_v.at[pl.ds(2 * p + 1, 1)], sem
                )
        for _ in range(18):
            pltpu.make_async_copy(
                table_hbm.at[pl.ds(0, 1)], pairs_v.at[pl.ds(0, 1)], sem
            ).wait()

        def fire(g):
            idxvec = idx_v[g, :]
            for j in range(_K // 2):
                code = 3 * idxvec[2 * j] + idxvec[2 * j + 1]
                pltpu.async_copy(
                    pairs_v.at[pl.ds(code * 2, 2)],
                    out_hbm.at[pl.ds(base + g * _K + 2 * j, 2)],
                    sem,
                )

        def drain(count):
            for _ in range(count):
                pltpu.make_async_copy(
                    pairs_v.at[pl.ds(0, 2)], out_hbm.at[pl.ds(base, 2)], sem
                ).wait()

        def step(g, _):
            fire(g)

            @pl.when(g >= 4)
            def _():
                drain(_K // 2)

            return 0

        lax.fori_loop(0, ch, step, 0)
        drain(4 * (_K // 2))

    return k


def kernel(modality_ids, table):
    b, s = modality_ids.shape
    n = b * s
    nc, ns = 2, 16
    nw = nc * ns
    ids3 = modality_ids.reshape(nw, (n // nw) // _K, _K).astype(jnp.int32)
    k = _make_sc_kernel(n, nc, ns)
    out = k(table, ids3)
    return out.reshape(b, s, _HIDDEN)


# speedup vs baseline: 1.1599x; 1.1599x over previous
import functools
import jax
import jax.numpy as jnp
from jax import lax
from jax.experimental import pallas as pl
from jax.experimental.pallas import tpu as pltpu
from jax.experimental.pallas import tpu_sc as plsc

_K = 16
_RING = 2


def _make_sc_kernel(n_total, n_rows, hidden, nc, ns):
    nw = nc * ns
    b_per_w = n_total // nw
    ch = b_per_w // _K

    mesh = plsc.VectorSubcoreMesh(
        core_axis_name="c", subcore_axis_name="s", num_cores=nc, num_subcores=ns
    )

    @functools.partial(
        pl.kernel,
        out_type=jax.ShapeDtypeStruct((n_total, hidden), jnp.float32),
        mesh=mesh,
        scratch_types=[
            pltpu.VMEM((n_rows, hidden), jnp.float32),
            pltpu.VMEM((ch, _K), jnp.int32),
            pltpu.SemaphoreType.DMA,
        ],
    )
    def k(table_hbm, idx_hbm, out_hbm, table_v, idx_v, sem):
        wid = lax.axis_index("s") * nc + lax.axis_index("c")
        base = wid * b_per_w
        pltpu.sync_copy(table_hbm, table_v)
        pltpu.sync_copy(idx_hbm.at[wid], idx_v)

        def fire(g):
            idxvec = idx_v[g, :]
            for r in range(_K):
                rowid = idxvec[r]
                pltpu.async_copy(
                    table_v.at[pl.ds(rowid, 1)],
                    out_hbm.at[pl.ds(base + g * _K + r, 1)],
                    sem,
                )

        def drain(count):
            for _ in range(count):
                pltpu.make_async_copy(
                    table_v.at[pl.ds(0, 1)], out_hbm.at[pl.ds(base, 1)], sem
                ).wait()

        def step(g, _):
            fire(g)

            @pl.when(g >= _RING)
            def _():
                drain(_K)

            return 0

        lax.fori_loop(0, ch, step, 0)
        drain(_RING * _K)

    return k


def kernel(modality_ids, table):
    b, s = modality_ids.shape
    n_rows, hidden = table.shape
    n = b * s
    nc, ns = 2, 16
    nw = nc * ns
    ids3 = modality_ids.reshape(nw, (n // nw) // _K, _K).astype(jnp.int32)
    k = _make_sc_kernel(n, n_rows, hidden, nc, ns)
    out = k(table, ids3)
    return out.reshape(b, s, hidden)
